# baseline (speedup 1.0000x reference)
import functools

import jax
import jax.numpy as jnp
from jax import lax
from jax.experimental import pallas as pl
from jax.experimental.pallas import tpu as pltpu
from jax.experimental.pallas import tpu_sc as plsc

_NUM_CORES = 2
_NUM_SUBCORES = 16
_NW = _NUM_CORES * _NUM_SUBCORES
_CHUNK = 128
_LANES = 16
_ROW = 128
_LB = 4096
_LB_BITS = _LB.bit_length() - 1
_HALF_BITS = _LB_BITS - 1
_HALF_MASK = (1 << _HALF_BITS) - 1


@functools.lru_cache(maxsize=None)
def _build_relayout(vocab, dim):
    grid = (vocab + _LB - 1) // _LB

    half = _LB // 2

    def merge(x):
        z = jnp.concatenate([x[:, :half], x[:, half:]], axis=0)
        return jnp.transpose(z, (1, 0))

    def body(a_ref, b_ref, oa_ref, ob_ref):
        oa_ref[...] = merge(a_ref[...])
        ob_ref[...] = merge(b_ref[...])

    in_spec = pl.BlockSpec((dim, _LB), lambda i: (0, i))
    out_spec = pl.BlockSpec((half, _ROW), lambda i: (i, 0))
    return pl.pallas_call(
        body,
        grid=(grid,),
        in_specs=[in_spec, in_spec],
        out_specs=[out_spec, out_spec],
        out_shape=[jax.ShapeDtypeStruct((grid * half, _ROW), jnp.float32)] * 2,
    )


@functools.lru_cache(maxsize=None)
def _build_gather(vocab, dim, batch):
    b_per_w = batch // _NW
    n_chunks = b_per_w // _CHUNK
    groups_per_chunk = _CHUNK // _LANES
    mesh = plsc.VectorSubcoreMesh(core_axis_name="c", subcore_axis_name="s")

    @functools.partial(
        pl.kernel,
        out_type=jax.ShapeDtypeStruct((batch,), jnp.float32),
        mesh=mesh,
        compiler_params=pltpu.CompilerParams(
            needs_layout_passes=False, use_tc_tiling_on_sc=True
        ),
        scratch_types=[
            pltpu.VMEM((b_per_w,), jnp.int32),
            pltpu.VMEM((b_per_w,), jnp.int32),
            pltpu.VMEM((b_per_w,), jnp.int32),
            pltpu.VMEM((b_per_w,), jnp.int32),
            pltpu.VMEM((_CHUNK, _ROW), jnp.float32),
            pltpu.VMEM((_CHUNK, _ROW), jnp.float32),
            pltpu.VMEM((_CHUNK, _ROW), jnp.float32),
            pltpu.VMEM((_CHUNK, _ROW), jnp.float32),
            pltpu.VMEM((b_per_w,), jnp.float32),
            pltpu.VMEM((b_per_w,), jnp.float32),
            pltpu.VMEM((b_per_w,), jnp.float32),
            pltpu.SemaphoreType.DMA,
            pltpu.SemaphoreType.DMA,
            pltpu.SemaphoreType.DMA,
        ],
    )
    def glove_kernel(wp_hbm, wtp_hbm, b_hbm, bt_hbm, i_hbm, j_hbm, out_hbm,
                     ii, jj, iih, jjh, wi_a, wi_b, wj_a, wj_b, bi, bj, ov,
                     sem_a, sem_b, bsem):
        wid = lax.axis_index("s") * _NUM_CORES + lax.axis_index("c")
        base = wid * b_per_w
        pltpu.sync_copy(i_hbm.at[pl.ds(base, b_per_w)], ii)
        pltpu.sync_copy(j_hbm.at[pl.ds(base, b_per_w)], jj)

        def shift_body(k, carry):
            sl = pl.ds(k * _LANES, _LANES)
            vi = ii[sl]
            vj = jj[sl]
            iih[sl] = ((vi >> _LB_BITS) << _HALF_BITS) | (vi & _HALF_MASK)
            jjh[sl] = ((vj >> _LB_BITS) << _HALF_BITS) | (vj & _HALF_MASK)
            return carry

        lax.fori_loop(0, b_per_w // _LANES, shift_body, 0)

        bias_copies = []
        for k in range(n_chunks):
            sl = pl.ds(k * _CHUNK, _CHUNK)
            bias_copies.append(
                pltpu.async_copy(b_hbm.at[ii.at[sl]], bi.at[sl], bsem))
            bias_copies.append(
                pltpu.async_copy(bt_hbm.at[jj.at[sl]], bj.at[sl], bsem))

        wi_bufs = [wi_a, wi_b]
        wj_bufs = [wj_a, wj_b]
        sems = [sem_a, sem_b]

        def fire(c):
            sl = pl.ds(c * _CHUNK, _CHUNK)
            s = sems[c % 2]
            return (
                pltpu.async_copy(wp_hbm.at[iih.at[sl]], wi_bufs[c % 2], s),
                pltpu.async_copy(wtp_hbm.at[jjh.at[sl]], wj_bufs[c % 2], s),
            )

        pending = {0: fire(0)}
        if n_chunks > 1:
            pending[1] = fire(1)
        for cp in bias_copies:
            cp.wait()

        for c in range(n_chunks):
            for cp in pending.pop(c):
                cp.wait()
            wi_buf = wi_bufs[c % 2]
            wj_buf = wj_bufs[c % 2]

            def group_body(g, carry, c=c, wi_buf=wi_buf, wj_buf=wj_buf):
                rows = g * _LANES + lax.iota(jnp.int32, _LANES)
                off = pl.ds(c * _CHUNK + g * _LANES, _LANES)
                ci0 = ((ii[off] >> _HALF_BITS) & 1) * dim
                cj0 = ((jj[off] >> _HALF_BITS) & 1) * dim
                acc = bi[off] + bj[off]
                for d in range(dim):
                    acc = acc + (
                        plsc.load_gather(wi_buf, [rows, ci0 + d])
                        * plsc.load_gather(wj_buf, [rows, cj0 + d]))
                ov[off] = acc
                return carry

            lax.fori_loop(0, groups_per_chunk, group_body, 0)
            if c + 2 < n_chunks:
                pending[c + 2] = fire(c + 2)

        pltpu.sync_copy(ov, out_hbm.at[pl.ds(base, b_per_w)])

    return glove_kernel


def kernel(W, W_tilde, b, b_tilde, i_idx, j_idx):
    vocab, dim = W.shape
    batch = i_idx.shape[0]
    wp, wtp = _build_relayout(vocab, dim)(W.T, W_tilde.T)
    return _build_gather(vocab, dim, batch)(
        wp,
        wtp,
        b.reshape(vocab),
        b_tilde.reshape(vocab),
        i_idx,
        j_idx,
    )


# speedup vs baseline: 1.1444x; 1.1444x over previous
import functools

import jax
import jax.numpy as jnp
from jax import lax
from jax.experimental import pallas as pl
from jax.experimental.pallas import tpu as pltpu
from jax.experimental.pallas import tpu_sc as plsc

_NUM_CORES = 2
_NUM_SUBCORES = 16
_NW = _NUM_CORES * _NUM_SUBCORES
_CHUNK = 128
_LANES = 16
_ROW = 128
_LB = 16384
_LB_BITS = _LB.bit_length() - 1
_HALF_BITS = _LB_BITS - 1
_HALF_MASK = (1 << _HALF_BITS) - 1


@functools.lru_cache(maxsize=None)
def _build_relayout(vocab, dim):
    grid = (vocab + _LB - 1) // _LB

    half = _LB // 2

    def merge(x):
        z = jnp.concatenate([x[:, :half], x[:, half:]], axis=0)
        return jnp.transpose(z, (1, 0))

    def body(a_ref, b_ref, oa_ref, ob_ref):
        oa_ref[...] = merge(a_ref[...])
        ob_ref[...] = merge(b_ref[...])

    in_spec = pl.BlockSpec((dim, _LB), lambda i: (0, i))
    out_spec = pl.BlockSpec((half, _ROW), lambda i: (i, 0))
    return pl.pallas_call(
        body,
        grid=(grid,),
        in_specs=[in_spec, in_spec],
        out_specs=[out_spec, out_spec],
        out_shape=[jax.ShapeDtypeStruct((grid * half, _ROW), jnp.float32)] * 2,
    )


@functools.lru_cache(maxsize=None)
def _build_gather(vocab, dim, batch):
    b_per_w = batch // _NW
    n_chunks = b_per_w // _CHUNK
    groups_per_chunk = _CHUNK // _LANES
    mesh = plsc.VectorSubcoreMesh(core_axis_name="c", subcore_axis_name="s")

    @functools.partial(
        pl.kernel,
        out_type=jax.ShapeDtypeStruct((batch,), jnp.float32),
        mesh=mesh,
        compiler_params=pltpu.CompilerParams(
            needs_layout_passes=False, use_tc_tiling_on_sc=True
        ),
        scratch_types=[
            pltpu.VMEM((b_per_w,), jnp.int32),
            pltpu.VMEM((b_per_w,), jnp.int32),
            pltpu.VMEM((b_per_w,), jnp.int32),
            pltpu.VMEM((b_per_w,), jnp.int32),
            pltpu.VMEM((_CHUNK, _ROW), jnp.float32),
            pltpu.VMEM((_CHUNK, _ROW), jnp.float32),
            pltpu.VMEM((_CHUNK, _ROW), jnp.float32),
            pltpu.VMEM((_CHUNK, _ROW), jnp.float32),
            pltpu.VMEM((b_per_w,), jnp.float32),
            pltpu.VMEM((b_per_w,), jnp.float32),
            pltpu.VMEM((b_per_w,), jnp.float32),
            pltpu.SemaphoreType.DMA,
            pltpu.SemaphoreType.DMA,
            pltpu.SemaphoreType.DMA,
        ],
    )
    def glove_kernel(wp_hbm, wtp_hbm, b_hbm, bt_hbm, i_hbm, j_hbm, out_hbm,
                     ii, jj, iih, jjh, wi_a, wi_b, wj_a, wj_b, bi, bj, ov,
                     sem_a, sem_b, bsem):
        wid = lax.axis_index("s") * _NUM_CORES + lax.axis_index("c")
        base = wid * b_per_w
        pltpu.sync_copy(i_hbm.at[pl.ds(base, b_per_w)], ii)
        pltpu.sync_copy(j_hbm.at[pl.ds(base, b_per_w)], jj)

        def shift_body(k, carry):
            sl = pl.ds(k * _LANES, _LANES)
            vi = ii[sl]
            vj = jj[sl]
            iih[sl] = ((vi >> _LB_BITS) << _HALF_BITS) | (vi & _HALF_MASK)
            jjh[sl] = ((vj >> _LB_BITS) << _HALF_BITS) | (vj & _HALF_MASK)
            return carry

        lax.fori_loop(0, b_per_w // _LANES, shift_body, 0)

        bias_copies = []
        for k in range(n_chunks):
            sl = pl.ds(k * _CHUNK, _CHUNK)
            bias_copies.append(
                pltpu.async_copy(b_hbm.at[ii.at[sl]], bi.at[sl], bsem))
            bias_copies.append(
                pltpu.async_copy(bt_hbm.at[jj.at[sl]], bj.at[sl], bsem))

        wi_bufs = [wi_a, wi_b]
        wj_bufs = [wj_a, wj_b]
        sems = [sem_a, sem_b]

        def fire(c):
            sl = pl.ds(c * _CHUNK, _CHUNK)
            s = sems[c % 2]
            return (
                pltpu.async_copy(wp_hbm.at[iih.at[sl]], wi_bufs[c % 2], s),
                pltpu.async_copy(wtp_hbm.at[jjh.at[sl]], wj_bufs[c % 2], s),
            )

        pending = {0: fire(0)}
        if n_chunks > 1:
            pending[1] = fire(1)
        for cp in bias_copies:
            cp.wait()

        for c in range(n_chunks):
            for cp in pending.pop(c):
                cp.wait()
            wi_buf = wi_bufs[c % 2]
            wj_buf = wj_bufs[c % 2]

            def group_body(g, carry, c=c, wi_buf=wi_buf, wj_buf=wj_buf):
                rows = g * _LANES + lax.iota(jnp.int32, _LANES)
                off = pl.ds(c * _CHUNK + g * _LANES, _LANES)
                ci0 = ((ii[off] >> _HALF_BITS) & 1) * dim
                cj0 = ((jj[off] >> _HALF_BITS) & 1) * dim
                acc = bi[off] + bj[off]
                for d in range(dim):
                    acc = acc + (
                        plsc.load_gather(wi_buf, [rows, ci0 + d])
                        * plsc.load_gather(wj_buf, [rows, cj0 + d]))
                ov[off] = acc
                return carry

            lax.fori_loop(0, groups_per_chunk, group_body, 0)
            if c + 2 < n_chunks:
                pending[c + 2] = fire(c + 2)

        pltpu.sync_copy(ov, out_hbm.at[pl.ds(base, b_per_w)])

    return glove_kernel


def kernel(W, W_tilde, b, b_tilde, i_idx, j_idx):
    vocab, dim = W.shape
    batch = i_idx.shape[0]
    wp, wtp = _build_relayout(vocab, dim)(W.T, W_tilde.T)
    return _build_gather(vocab, dim, batch)(
        wp,
        wtp,
        b.reshape(vocab),
        b_tilde.reshape(vocab),
        i_idx,
        j_idx,
    )


# speedup vs baseline: 1.3961x; 1.2200x over previous
import functools

import jax
import jax.numpy as jnp
from jax import lax
from jax.experimental import pallas as pl
from jax.experimental.pallas import tpu as pltpu
from jax.experimental.pallas import tpu_sc as plsc

_NUM_CORES = 2
_NUM_SUBCORES = 16
_NW = _NUM_CORES * _NUM_SUBCORES
_CHUNK = 128
_LANES = 16
_ROW = 128
_LB = 16384
_LB_BITS = _LB.bit_length() - 1
_QUART_MASK = (_LB // 4) - 1


@functools.lru_cache(maxsize=None)
def _build_relayout(vocab, dim):
    grid = (vocab + _LB - 1) // _LB

    half = _LB // 2
    quart = _LB // 4

    def merge(x):
        z = jnp.concatenate([x[:, :half], x[:, half:]], axis=0)
        zt = jnp.transpose(z, (1, 0))
        w = lax.bitcast_convert_type(zt, jnp.int32) + 0x8000
        lo = (w[:quart] >> 16) & 0xFFFF
        hi = w[quart:] & -65536
        return hi | lo

    def body(a_ref, b_ref, oa_ref, ob_ref):
        oa_ref[...] = merge(a_ref[...])
        ob_ref[...] = merge(b_ref[...])

    in_spec = pl.BlockSpec((dim, _LB), lambda i: (0, i))
    out_spec = pl.BlockSpec((quart, _ROW), lambda i: (i, 0))
    return pl.pallas_call(
        body,
        grid=(grid,),
        in_specs=[in_spec, in_spec],
        out_specs=[out_spec, out_spec],
        out_shape=[jax.ShapeDtypeStruct((grid * quart, _ROW), jnp.int32)] * 2,
    )


@functools.lru_cache(maxsize=None)
def _build_gather(vocab, dim, batch):
    b_per_w = batch // _NW
    n_chunks = b_per_w // _CHUNK
    groups_per_chunk = _CHUNK // _LANES
    mesh = plsc.VectorSubcoreMesh(core_axis_name="c", subcore_axis_name="s")

    @functools.partial(
        pl.kernel,
        out_type=jax.ShapeDtypeStruct((batch,), jnp.float32),
        mesh=mesh,
        compiler_params=pltpu.CompilerParams(
            needs_layout_passes=False, use_tc_tiling_on_sc=True
        ),
        scratch_types=[
            pltpu.VMEM((b_per_w,), jnp.int32),
            pltpu.VMEM((b_per_w,), jnp.int32),
            pltpu.VMEM((b_per_w,), jnp.int32),
            pltpu.VMEM((b_per_w,), jnp.int32),
            pltpu.VMEM((_CHUNK, _ROW), jnp.int32),
            pltpu.VMEM((_CHUNK, _ROW), jnp.int32),
            pltpu.VMEM((_CHUNK, _ROW), jnp.int32),
            pltpu.VMEM((_CHUNK, _ROW), jnp.int32),
            pltpu.VMEM((b_per_w,), jnp.float32),
            pltpu.VMEM((b_per_w,), jnp.float32),
            pltpu.VMEM((b_per_w,), jnp.float32),
            pltpu.SemaphoreType.DMA,
            pltpu.SemaphoreType.DMA,
            pltpu.SemaphoreType.DMA,
        ],
    )
    def glove_kernel(wp_hbm, wtp_hbm, b_hbm, bt_hbm, i_hbm, j_hbm, out_hbm,
                     ii, jj, iih, jjh, wi_a, wi_b, wj_a, wj_b, bi, bj, ov,
                     sem_a, sem_b, bsem):
        wid = lax.axis_index("s") * _NUM_CORES + lax.axis_index("c")
        base = wid * b_per_w
        pltpu.sync_copy(i_hbm.at[pl.ds(base, b_per_w)], ii)
        pltpu.sync_copy(j_hbm.at[pl.ds(base, b_per_w)], jj)

        def shift_body(k, carry):
            sl = pl.ds(k * _LANES, _LANES)
            vi = ii[sl]
            vj = jj[sl]
            iih[sl] = ((vi >> _LB_BITS) << (_LB_BITS - 2)) | (vi & _QUART_MASK)
            jjh[sl] = ((vj >> _LB_BITS) << (_LB_BITS - 2)) | (vj & _QUART_MASK)
            return carry

        lax.fori_loop(0, b_per_w // _LANES, shift_body, 0)

        bias_copies = []
        for k in range(n_chunks):
            sl = pl.ds(k * _CHUNK, _CHUNK)
            bias_copies.append(
                pltpu.async_copy(b_hbm.at[ii.at[sl]], bi.at[sl], bsem))
            bias_copies.append(
                pltpu.async_copy(bt_hbm.at[jj.at[sl]], bj.at[sl], bsem))

        wi_bufs = [wi_a, wi_b]
        wj_bufs = [wj_a, wj_b]
        sems = [sem_a, sem_b]

        def fire(c):
            sl = pl.ds(c * _CHUNK, _CHUNK)
            s = sems[c % 2]
            return (
                pltpu.async_copy(wp_hbm.at[iih.at[sl]], wi_bufs[c % 2], s),
                pltpu.async_copy(wtp_hbm.at[jjh.at[sl]], wj_bufs[c % 2], s),
            )

        pending = {0: fire(0)}
        if n_chunks > 1:
            pending[1] = fire(1)
        for cp in bias_copies:
            cp.wait()

        for c in range(n_chunks):
            for cp in pending.pop(c):
                cp.wait()
            wi_buf = wi_bufs[c % 2]
            wj_buf = wj_bufs[c % 2]

            def group_body(g, carry, c=c, wi_buf=wi_buf, wj_buf=wj_buf):
                rows = g * _LANES + lax.iota(jnp.int32, _LANES)
                off = pl.ds(c * _CHUNK + g * _LANES, _LANES)
                vi = ii[off]
                vj = jj[off]
                ci0 = ((vi >> (_LB_BITS - 1)) & 1) * dim
                cj0 = ((vj >> (_LB_BITS - 1)) & 1) * dim
                shi = (1 - ((vi >> (_LB_BITS - 2)) & 1)) * 16
                shj = (1 - ((vj >> (_LB_BITS - 2)) & 1)) * 16
                acc = bi[off] + bj[off]
                for d in range(dim):
                    wi_v = (plsc.load_gather(wi_buf, [rows, ci0 + d]) << shi) & -65536
                    wj_v = (plsc.load_gather(wj_buf, [rows, cj0 + d]) << shj) & -65536
                    acc = acc + (plsc.bitcast(wi_v, jnp.float32)
                                 * plsc.bitcast(wj_v, jnp.float32))
                ov[off] = acc
                return carry

            lax.fori_loop(0, groups_per_chunk, group_body, 0)
            if c + 2 < n_chunks:
                pending[c + 2] = fire(c + 2)

        pltpu.sync_copy(ov, out_hbm.at[pl.ds(base, b_per_w)])

    return glove_kernel


def kernel(W, W_tilde, b, b_tilde, i_idx, j_idx):
    vocab, dim = W.shape
    batch = i_idx.shape[0]
    wp, wtp = _build_relayout(vocab, dim)(W.T, W_tilde.T)
    return _build_gather(vocab, dim, batch)(
        wp,
        wtp,
        b.reshape(vocab),
        b_tilde.reshape(vocab),
        i_idx,
        j_idx,
    )


# speedup vs baseline: 1.4171x; 1.0150x over previous
import functools

import jax
import jax.numpy as jnp
from jax import lax
from jax.experimental import pallas as pl
from jax.experimental.pallas import tpu as pltpu
from jax.experimental.pallas import tpu_sc as plsc

_NUM_CORES = 2
_NUM_SUBCORES = 16
_NW = _NUM_CORES * _NUM_SUBCORES
_CHUNK = 128
_LANES = 16
_ROW = 128
_LB = 32768
_LB_BITS = _LB.bit_length() - 1
_QUART_MASK = (_LB // 4) - 1


@functools.lru_cache(maxsize=None)
def _build_relayout(vocab, dim):
    grid = (vocab + _LB - 1) // _LB

    half = _LB // 2
    quart = _LB // 4

    def merge(x):
        z = jnp.concatenate([x[:, :half], x[:, half:]], axis=0)
        zt = jnp.transpose(z, (1, 0))
        w = lax.bitcast_convert_type(zt, jnp.int32) + 0x8000
        lo = (w[:quart] >> 16) & 0xFFFF
        hi = w[quart:] & -65536
        return hi | lo

    def body(a_ref, b_ref, oa_ref, ob_ref):
        oa_ref[...] = merge(a_ref[...])
        ob_ref[...] = merge(b_ref[...])

    in_spec = pl.BlockSpec((dim, _LB), lambda i: (0, i))
    out_spec = pl.BlockSpec((quart, _ROW), lambda i: (i, 0))
    return pl.pallas_call(
        body,
        grid=(grid,),
        in_specs=[in_spec, in_spec],
        out_specs=[out_spec, out_spec],
        out_shape=[jax.ShapeDtypeStruct((grid * quart, _ROW), jnp.int32)] * 2,
    )


@functools.lru_cache(maxsize=None)
def _build_gather(vocab, dim, batch):
    b_per_w = batch // _NW
    n_chunks = b_per_w // _CHUNK
    groups_per_chunk = _CHUNK // _LANES
    mesh = plsc.VectorSubcoreMesh(core_axis_name="c", subcore_axis_name="s")

    @functools.partial(
        pl.kernel,
        out_type=jax.ShapeDtypeStruct((batch,), jnp.float32),
        mesh=mesh,
        compiler_params=pltpu.CompilerParams(
            needs_layout_passes=False, use_tc_tiling_on_sc=True
        ),
        scratch_types=[
            pltpu.VMEM((b_per_w,), jnp.int32),
            pltpu.VMEM((b_per_w,), jnp.int32),
            pltpu.VMEM((b_per_w,), jnp.int32),
            pltpu.VMEM((b_per_w,), jnp.int32),
            pltpu.VMEM((_CHUNK, _ROW), jnp.int32),
            pltpu.VMEM((_CHUNK, _ROW), jnp.int32),
            pltpu.VMEM((_CHUNK, _ROW), jnp.int32),
            pltpu.VMEM((_CHUNK, _ROW), jnp.int32),
            pltpu.VMEM((b_per_w,), jnp.float32),
            pltpu.VMEM((b_per_w,), jnp.float32),
            pltpu.VMEM((b_per_w,), jnp.float32),
            pltpu.SemaphoreType.DMA,
            pltpu.SemaphoreType.DMA,
            pltpu.SemaphoreType.DMA,
        ],
    )
    def glove_kernel(wp_hbm, wtp_hbm, b_hbm, bt_hbm, i_hbm, j_hbm, out_hbm,
                     ii, jj, iih, jjh, wi_a, wi_b, wj_a, wj_b, bi, bj, ov,
                     sem_a, sem_b, bsem):
        wid = lax.axis_index("s") * _NUM_CORES + lax.axis_index("c")
        base = wid * b_per_w
        pltpu.sync_copy(i_hbm.at[pl.ds(base, b_per_w)], ii)
        pltpu.sync_copy(j_hbm.at[pl.ds(base, b_per_w)], jj)

        def shift_body(k, carry):
            sl = pl.ds(k * _LANES, _LANES)
            vi = ii[sl]
            vj = jj[sl]
            iih[sl] = ((vi >> _LB_BITS) << (_LB_BITS - 2)) | (vi & _QUART_MASK)
            jjh[sl] = ((vj >> _LB_BITS) << (_LB_BITS - 2)) | (vj & _QUART_MASK)
            return carry

        lax.fori_loop(0, b_per_w // _LANES, shift_body, 0)

        bias_copies = []
        for k in range(n_chunks):
            sl = pl.ds(k * _CHUNK, _CHUNK)
            bias_copies.append(
                pltpu.async_copy(b_hbm.at[ii.at[sl]], bi.at[sl], bsem))
            bias_copies.append(
                pltpu.async_copy(bt_hbm.at[jj.at[sl]], bj.at[sl], bsem))

        wi_bufs = [wi_a, wi_b]
        wj_bufs = [wj_a, wj_b]
        sems = [sem_a, sem_b]

        def fire(c):
            sl = pl.ds(c * _CHUNK, _CHUNK)
            s = sems[c % 2]
            return (
                pltpu.async_copy(wp_hbm.at[iih.at[sl]], wi_bufs[c % 2], s),
                pltpu.async_copy(wtp_hbm.at[jjh.at[sl]], wj_bufs[c % 2], s),
            )

        pending = {0: fire(0)}
        if n_chunks > 1:
            pending[1] = fire(1)
        for cp in bias_copies:
            cp.wait()

        for c in range(n_chunks):
            for cp in pending.pop(c):
                cp.wait()
            wi_buf = wi_bufs[c % 2]
            wj_buf = wj_bufs[c % 2]

            def group_body(g, carry, c=c, wi_buf=wi_buf, wj_buf=wj_buf):
                rows = g * _LANES + lax.iota(jnp.int32, _LANES)
                off = pl.ds(c * _CHUNK + g * _LANES, _LANES)
                vi = ii[off]
                vj = jj[off]
                ci0 = ((vi >> (_LB_BITS - 1)) & 1) * dim
                cj0 = ((vj >> (_LB_BITS - 1)) & 1) * dim
                shi = (1 - ((vi >> (_LB_BITS - 2)) & 1)) * 16
                shj = (1 - ((vj >> (_LB_BITS - 2)) & 1)) * 16
                acc = bi[off] + bj[off]
                for d in range(dim):
                    wi_v = (plsc.load_gather(wi_buf, [rows, ci0 + d]) << shi) & -65536
                    wj_v = (plsc.load_gather(wj_buf, [rows, cj0 + d]) << shj) & -65536
                    acc = acc + (plsc.bitcast(wi_v, jnp.float32)
                                 * plsc.bitcast(wj_v, jnp.float32))
                ov[off] = acc
                return carry

            lax.fori_loop(0, groups_per_chunk, group_body, 0)
            if c + 2 < n_chunks:
                pending[c + 2] = fire(c + 2)

        pltpu.sync_copy(ov, out_hbm.at[pl.ds(base, b_per_w)])

    return glove_kernel


def kernel(W, W_tilde, b, b_tilde, i_idx, j_idx):
    vocab, dim = W.shape
    batch = i_idx.shape[0]
    wp, wtp = _build_relayout(vocab, dim)(W.T, W_tilde.T)
    return _build_gather(vocab, dim, batch)(
        wp,
        wtp,
        b.reshape(vocab),
        b_tilde.reshape(vocab),
        i_idx,
        j_idx,
    )
